# baseline (speedup 1.0000x reference)
import jax
import jax.numpy as jnp
from jax.experimental import pallas as pl
from jax.experimental.pallas import tpu as pltpu

N = 2048
IMG_FLAT = 3 * 32 * 32
BR1 = 256
BR2 = 256
HALF = N // 2

_PARALLEL1 = pltpu.CompilerParams(dimension_semantics=("parallel",))


def _build_body(pa_ref, x_ref, wenc_ref, a_ref, p0_ref):
    a_ref[...] = (
        (pa_ref[0] + pa_ref[1] + pa_ref[2]) * (1.0 / 3.0)
    ).astype(jnp.bfloat16)
    p0_ref[...] = jnp.tanh(
        jnp.dot(
            x_ref[...].astype(jnp.bfloat16),
            wenc_ref[...].astype(jnp.bfloat16),
            preferred_element_type=jnp.float32,
        )
    )


def _layers_body(atop_ref, abot_ref, p0_ref,
                 w0, w1, w2, w3, w4, w5, w6, w7, pfin_ref):
    p = p0_ref[...]
    for w_ref in (w0, w1, w2, w3, w4, w5, w6, w7):
        w = w_ref[...]
        hp = jnp.dot(p, w, preferred_element_type=jnp.float32)
        hpb = hp.astype(jnp.bfloat16)
        agg_t = jnp.dot(atop_ref[...], hpb, preferred_element_type=jnp.float32)
        agg_b = jnp.dot(abot_ref[...], hpb, preferred_element_type=jnp.float32)
        p = jnp.tanh(hp + jnp.concatenate([agg_t, agg_b], axis=0))
    pfin_ref[...] = p.astype(jnp.bfloat16)


def _decode_body(p_ref, wdec_ref, out_ref):
    out_ref[...] = jnp.dot(
        p_ref[...],
        wdec_ref[...].astype(jnp.bfloat16),
        preferred_element_type=jnp.float32,
    )


def kernel(imgs, primal_adjacency_tensor, dual_adjacency_tensor, dual_nodes, params):
    del dual_adjacency_tensor, dual_nodes
    n = imgs.shape[0]
    imgs_flat = imgs.reshape(n, IMG_FLAT)

    ws = [params["Wp_enc_%d" % i] for i in range(4)] + [
        params["Wp_dec_%d" % i] for i in range(4)
    ]

    a_mean, p0 = pl.pallas_call(
        _build_body,
        grid=(N // BR1,),
        in_specs=[
            pl.BlockSpec((3, BR1, N), lambda i: (0, i, 0)),
            pl.BlockSpec((BR1, IMG_FLAT), lambda i: (i, 0)),
            pl.BlockSpec((IMG_FLAT, 64), lambda i: (0, 0)),
        ],
        out_specs=[
            pl.BlockSpec((BR1, N), lambda i: (i, 0)),
            pl.BlockSpec((BR1, 64), lambda i: (i, 0)),
        ],
        out_shape=[
            jax.ShapeDtypeStruct((N, N), jnp.bfloat16),
            jax.ShapeDtypeStruct((n, 64), jnp.float32),
        ],
        compiler_params=_PARALLEL1,
    )(primal_adjacency_tensor, imgs_flat, params["W_img_enc"])

    pfin = pl.pallas_call(
        _layers_body,
        grid=(1,),
        in_specs=[
            pl.BlockSpec((HALF, N), lambda k: (0, 0)),
            pl.BlockSpec((HALF, N), lambda k: (1, 0)),
            pl.BlockSpec((n, 64), lambda k: (0, 0)),
        ]
        + [pl.BlockSpec(w.shape, lambda k: (0, 0)) for w in ws],
        out_specs=pl.BlockSpec((n, 64), lambda k: (0, 0)),
        out_shape=jax.ShapeDtypeStruct((n, 64), jnp.bfloat16),
    )(a_mean, a_mean, p0, *ws)

    recon = pl.pallas_call(
        _decode_body,
        grid=(n // BR2,),
        in_specs=[
            pl.BlockSpec((BR2, 64), lambda k: (k, 0)),
            pl.BlockSpec((64, IMG_FLAT), lambda k: (0, 0)),
        ],
        out_specs=pl.BlockSpec((BR2, IMG_FLAT), lambda k: (k, 0)),
        out_shape=jax.ShapeDtypeStruct((n, IMG_FLAT), jnp.float32),
        compiler_params=_PARALLEL1,
    )(pfin, params["W_img_dec"])

    return recon.reshape(imgs.shape)


# speedup vs baseline: 1.0878x; 1.0878x over previous
import jax
import jax.numpy as jnp
from jax.experimental import pallas as pl
from jax.experimental.pallas import tpu as pltpu

N = 2048
IMG_FLAT = 3 * 32 * 32
BR = 256
NBLK = N // BR


def _body(pa0_ref, pa1_ref, pa2_ref, x_ref, wenc_ref, wdec_ref,
          w0, w1, w2, w3, w4, w5, w6, w7,
          out_ref, a_s, p0_s, pfin_s):
    j = pl.program_id(0)

    @pl.when(j < NBLK)
    def _build():
        a_s[pl.ds(j * BR, BR), :] = (
            (pa0_ref[0] + pa1_ref[0] + pa2_ref[0]) * (1.0 / 3.0)
        ).astype(jnp.bfloat16)
        p0_s[pl.ds(j * BR, BR), :] = jnp.tanh(
            jnp.dot(
                x_ref[...].astype(jnp.bfloat16),
                wenc_ref[...].astype(jnp.bfloat16),
                preferred_element_type=jnp.float32,
            )
        )

    @pl.when(j == NBLK)
    def _layers():
        A = a_s[...]
        p = p0_s[...]
        for w_ref in (w0, w1, w2, w3, w4, w5, w6, w7):
            w = w_ref[...]
            hp = jnp.dot(p, w, preferred_element_type=jnp.float32)
            agg = jnp.dot(
                A, hp.astype(jnp.bfloat16), preferred_element_type=jnp.float32
            )
            p = jnp.tanh(hp + agg)
        pfin_s[...] = p

    @pl.when(j >= NBLK)
    def _decode():
        blk = j - NBLK
        out_ref[...] = jnp.dot(
            pfin_s[pl.ds(blk * BR, BR), :].astype(jnp.bfloat16),
            wdec_ref[...].astype(jnp.bfloat16),
            preferred_element_type=jnp.float32,
        )


def kernel(imgs, primal_adjacency_tensor, dual_adjacency_tensor, dual_nodes, params):
    del dual_adjacency_tensor, dual_nodes
    n = imgs.shape[0]
    imgs_flat = imgs.reshape(n, IMG_FLAT)

    ws = [params["Wp_enc_%d" % i] for i in range(4)] + [
        params["Wp_dec_%d" % i] for i in range(4)
    ]

    recon_call = pl.pallas_call(
        _body,
        grid=(2 * NBLK,),
        in_specs=[
            pl.BlockSpec((1, BR, N), lambda j: (0, jnp.minimum(j, NBLK - 1), 0)),
            pl.BlockSpec((1, BR, N), lambda j: (1, jnp.minimum(j, NBLK - 1), 0)),
            pl.BlockSpec((1, BR, N), lambda j: (2, jnp.minimum(j, NBLK - 1), 0)),
            pl.BlockSpec((BR, IMG_FLAT), lambda j: (jnp.minimum(j, NBLK - 1), 0)),
            pl.BlockSpec((IMG_FLAT, 64), lambda j: (0, 0)),
            pl.BlockSpec((64, IMG_FLAT), lambda j: (0, 0)),
        ]
        + [pl.BlockSpec(w.shape, lambda j: (0, 0)) for w in ws],
        out_specs=pl.BlockSpec(
            (BR, IMG_FLAT), lambda j: (jnp.maximum(j - NBLK, 0), 0)
        ),
        out_shape=jax.ShapeDtypeStruct((n, IMG_FLAT), jnp.float32),
        scratch_shapes=[
            pltpu.VMEM((N, N), jnp.bfloat16),
            pltpu.VMEM((N, 64), jnp.float32),
            pltpu.VMEM((N, 64), jnp.float32),
        ],
    )
    pa = primal_adjacency_tensor
    recon = recon_call(pa, pa, pa, imgs_flat,
                       params["W_img_enc"], params["W_img_dec"], *ws)

    return recon.reshape(imgs.shape)


# speedup vs baseline: 1.0931x; 1.0048x over previous
import jax
import jax.numpy as jnp
from jax.experimental import pallas as pl
from jax.experimental.pallas import tpu as pltpu

N = 2048
IMG_FLAT = 3 * 32 * 32
BR = 256
NBLK = N // BR


def _body(pa0_ref, pa1_ref, pa2_ref, x_ref, wenc_ref, wdec_ref,
          w0, w1, w2, w3, w4, w5, w6, w7,
          out_ref, a_s, p0_s, pfin_s):
    j = pl.program_id(0)

    @pl.when(j < NBLK)
    def _build():
        a_s[pl.ds(j * BR, BR), :] = (
            (pa0_ref[0] + pa1_ref[0] + pa2_ref[0]) * (1.0 / 3.0)
        ).astype(jnp.bfloat16)
        p0_s[pl.ds(j * BR, BR), :] = jnp.tanh(
            jnp.dot(
                x_ref[...].astype(jnp.bfloat16),
                wenc_ref[...].astype(jnp.bfloat16),
                preferred_element_type=jnp.float32,
            )
        )

    @pl.when(j == NBLK)
    def _layers():
        A = a_s[...]
        p = p0_s[...]
        for w_ref in (w0, w1, w2, w3, w4, w5, w6):
            w = w_ref[...]
            hp = jnp.dot(p, w, preferred_element_type=jnp.float32)
            agg = jnp.dot(
                A, hp.astype(jnp.bfloat16), preferred_element_type=jnp.float32
            )
            p = jnp.tanh(hp + agg)
        pfin_s[...] = jnp.dot(p, w7[...], preferred_element_type=jnp.float32)

    @pl.when(j >= NBLK)
    def _decode():
        blk = j - NBLK
        hp8 = pfin_s[...]
        agg8 = jnp.dot(
            a_s[pl.ds(blk * BR, BR), :],
            hp8.astype(jnp.bfloat16),
            preferred_element_type=jnp.float32,
        )
        p_blk = jnp.tanh(pfin_s[pl.ds(blk * BR, BR), :] + agg8)
        out_ref[...] = jnp.dot(
            p_blk.astype(jnp.bfloat16),
            wdec_ref[...].astype(jnp.bfloat16),
            preferred_element_type=jnp.float32,
        )


def kernel(imgs, primal_adjacency_tensor, dual_adjacency_tensor, dual_nodes, params):
    del dual_adjacency_tensor, dual_nodes
    n = imgs.shape[0]
    imgs_flat = imgs.reshape(n, IMG_FLAT)

    ws = [params["Wp_enc_%d" % i] for i in range(4)] + [
        params["Wp_dec_%d" % i] for i in range(4)
    ]

    recon_call = pl.pallas_call(
        _body,
        grid=(2 * NBLK,),
        in_specs=[
            pl.BlockSpec((1, BR, N), lambda j: (0, jnp.minimum(j, NBLK - 1), 0)),
            pl.BlockSpec((1, BR, N), lambda j: (1, jnp.minimum(j, NBLK - 1), 0)),
            pl.BlockSpec((1, BR, N), lambda j: (2, jnp.minimum(j, NBLK - 1), 0)),
            pl.BlockSpec((BR, IMG_FLAT), lambda j: (jnp.minimum(j, NBLK - 1), 0)),
            pl.BlockSpec((IMG_FLAT, 64), lambda j: (0, 0)),
            pl.BlockSpec((64, IMG_FLAT), lambda j: (0, 0)),
        ]
        + [pl.BlockSpec(w.shape, lambda j: (0, 0)) for w in ws],
        out_specs=pl.BlockSpec(
            (BR, IMG_FLAT), lambda j: (jnp.maximum(j - NBLK, 0), 0)
        ),
        out_shape=jax.ShapeDtypeStruct((n, IMG_FLAT), jnp.float32),
        scratch_shapes=[
            pltpu.VMEM((N, N), jnp.bfloat16),
            pltpu.VMEM((N, 64), jnp.float32),
            pltpu.VMEM((N, 64), jnp.float32),
        ],
    )
    pa = primal_adjacency_tensor
    recon = recon_call(pa, pa, pa, imgs_flat,
                       params["W_img_enc"], params["W_img_dec"], *ws)

    return recon.reshape(imgs.shape)
